# baseline (speedup 1.0000x reference)
import functools

import jax
import jax.numpy as jnp
from jax import lax
from jax.experimental import pallas as pl
from jax.experimental.pallas import tpu as pltpu
from jax.experimental.pallas import tpu_sc as plsc

N = 10000
D = 128
E = 320000
CH = 128
NC = 2
NS = 16
NW = NC * NS
EPT = E // NW
CPT = 80
PADE = CPT * CH - EPT
IB = 16
NPAD = 10240
DUMP = NPAD
RPT = NPAD // NS
L = 16


def _agg_body(x_hbm, srcs_hbm, dsts_hbm, p_hbm, cnt_hbm, src_v, dst_v,
              buf0, buf1, cnt_v, acc_sh, g0, g1):
    c = lax.axis_index("c")
    s = lax.axis_index("s")
    wid = c * NS + s

    def zrow(i, carry):
        buf0[i // 8, pl.ds((i % 8) * L, L)] = jnp.zeros((L,), jnp.float32)
        return carry
    lax.fori_loop(0, CH * 8, zrow, 0)
    for j in range(RPT // CH):
        off = pl.multiple_of(s * RPT + j * CH, 8)
        pltpu.sync_copy(buf0, acc_sh.at[pl.ds(off, CH)])

    def zc(i, carry):
        cnt_v[pl.ds(i * L, L)] = jnp.zeros((L,), jnp.float32)
        return carry
    lax.fori_loop(0, (NPAD + L) // L, zc, 0)
    plsc.subcore_barrier()

    ones = jnp.ones((L,), jnp.float32)

    def counts(k):
        for i in range(CH // L):
            plsc.addupdate_scatter(cnt_v, [dst_v[k, pl.ds(i * L, L)]], ones)

    def wait(sem):
        pltpu.make_async_copy(x_hbm.at[pl.ds(0, CH)], buf0, sem).wait()

    def gissue(k, buf, sem):
        pltpu.async_copy(x_hbm.at[src_v.at[k]], buf, sem)

    for blk in range(CPT // IB):
        pltpu.sync_copy(srcs_hbm.at[wid, pl.ds(blk * IB, IB)], src_v)
        pltpu.sync_copy(dsts_hbm.at[wid, pl.ds(blk * IB, IB)], dst_v)
        gissue(0, buf0, g0)

        def pair(i, carry):
            a = 2 * i
            wait(g0)
            gissue(a + 1, buf1, g1)
            counts(a)
            pltpu.sync_copy(buf0, acc_sh.at[dst_v.at[a]], add=True)
            gissue(a + 2, buf0, g0)
            wait(g1)
            counts(a + 1)
            pltpu.sync_copy(buf1, acc_sh.at[dst_v.at[a + 1]], add=True)
            return carry
        lax.fori_loop(0, IB // 2 - 1, pair, 0)

        wait(g0)
        gissue(IB - 1, buf1, g1)
        counts(IB - 2)
        pltpu.sync_copy(buf0, acc_sh.at[dst_v.at[IB - 2]], add=True)
        wait(g1)
        counts(IB - 1)
        pltpu.sync_copy(buf1, acc_sh.at[dst_v.at[IB - 1]], add=True)

    plsc.subcore_barrier()
    for j in range(RPT // CH):
        off = pl.multiple_of(s * RPT + j * CH, 8)
        pltpu.sync_copy(acc_sh.at[pl.ds(off, CH)], buf0)
        pltpu.sync_copy(buf0, p_hbm.at[c, pl.ds(off, CH)])
    pltpu.sync_copy(cnt_v.at[pl.ds(0, NPAD)], cnt_hbm.at[wid])


_AGG_CNT = pl.kernel(
    _agg_body,
    out_type=[
        jax.ShapeDtypeStruct((NC, NPAD, D), jnp.float32),
        jax.ShapeDtypeStruct((NW, NPAD), jnp.float32),
    ],
    mesh=plsc.VectorSubcoreMesh(core_axis_name="c", subcore_axis_name="s"),
    scratch_types=[
        pltpu.VMEM((IB, CH), jnp.int32),
        pltpu.VMEM((IB, CH), jnp.int32),
        pltpu.VMEM((CH, D), jnp.float32),
        pltpu.VMEM((CH, D), jnp.float32),
        pltpu.VMEM((NPAD + L,), jnp.float32),
        pltpu.VMEM_SHARED((NPAD + 8, D), jnp.float32),
        pltpu.SemaphoreType.DMA,
        pltpu.SemaphoreType.DMA,
    ],
    compiler_params=pltpu.CompilerParams(
        use_tc_tiling_on_sc=False, needs_layout_passes=False),
)

BN = 1024


def _layer_body(p_ref, c_ref, x_ref, wl_ref, wr_ref, b_ref, f_ref, o_ref):
    psum = p_ref[0] + p_ref[1]
    cnt_col = lax.dot_general(
        c_ref[...], jnp.ones((NW, 1), jnp.float32),
        (((0,), (0,)), ((), ())),
        preferred_element_type=jnp.float32,
        precision=lax.Precision.HIGHEST,
    )
    mean = psum / jnp.maximum(cnt_col, 1.0)
    acc = (
        jnp.dot(mean, wl_ref[...], preferred_element_type=jnp.float32,
                precision=lax.Precision.HIGHEST)
        + jnp.dot(x_ref[...], wr_ref[...], preferred_element_type=jnp.float32,
                  precision=lax.Precision.HIGHEST)
        + b_ref[...]
    )
    o_ref[...] = jnp.maximum(acc, acc * f_ref[...])


_LAYER = pl.pallas_call(
    _layer_body,
    grid=(NPAD // BN,),
    in_specs=[
        pl.BlockSpec((NC, BN, D), lambda i: (0, i, 0)),
        pl.BlockSpec((NW, BN), lambda i: (0, i)),
        pl.BlockSpec((BN, D), lambda i: (i, 0)),
        pl.BlockSpec((D, D), lambda i: (0, 0)),
        pl.BlockSpec((D, D), lambda i: (0, 0)),
        pl.BlockSpec((1, D), lambda i: (0, 0)),
        pl.BlockSpec((1, D), lambda i: (0, 0)),
    ],
    out_specs=pl.BlockSpec((BN, D), lambda i: (i, 0)),
    out_shape=jax.ShapeDtypeStruct((N, D), jnp.float32),
)


def kernel(x, edge_index, Wl1, Wr1, b1, Wl2, Wr2, b2):
    src3 = jnp.pad(edge_index[0].reshape(NW, EPT),
                   ((0, 0), (0, PADE))).reshape(NW, CPT, CH)
    dst3 = jnp.pad(edge_index[1].reshape(NW, EPT), ((0, 0), (0, PADE)),
                   constant_values=DUMP).reshape(NW, CPT, CH)
    Wl = jnp.stack([Wl1, Wl2])
    Wr = jnp.stack([Wr1, Wr2])
    bb = jnp.stack([b1.reshape(1, D), b2.reshape(1, D)])
    ff = jnp.stack([jnp.zeros((1, D), jnp.float32),
                    jnp.ones((1, D), jnp.float32)])

    def step(feat, ws):
        wl, wr, b, f = ws
        p, cnt = _AGG_CNT(feat, src3, dst3)
        return _LAYER(p, cnt, feat, wl, wr, b, f), 0.0

    out, _ = lax.scan(step, x, (Wl, Wr, bb, ff))
    return out


# speedup vs baseline: 2.4899x; 2.4899x over previous
import functools

import jax
import jax.numpy as jnp
from jax import lax
from jax.experimental import pallas as pl
from jax.experimental.pallas import tpu as pltpu
from jax.experimental.pallas import tpu_sc as plsc

N = 10000
D = 128
E = 320000
CH = 80
NC = 2
NS = 16
NW = NC * NS
EPT = E // NW
CPT = 125
PADE = CPT * CH - EPT
IB = 25
NPAD = 10240
DUMP = NPAD
RPT = NPAD // NS
L = 16


def _agg_body(x_hbm, srcs_hbm, dsts_hbm, p_hbm, cnt_hbm, src_v, dst_v,
              buf0, buf1, cnt_v, acc_sh, g0, g1):
    c = lax.axis_index("c")
    s = lax.axis_index("s")
    wid = c * NS + s

    def zrow(i, carry):
        buf0[i // 8, pl.ds((i % 8) * L, L)] = jnp.zeros((L,), jnp.float32)
        return carry
    lax.fori_loop(0, CH * 8, zrow, 0)
    for j in range(RPT // CH):
        off = pl.multiple_of(s * RPT + j * CH, 8)
        pltpu.sync_copy(buf0, acc_sh.at[pl.ds(off, CH)])

    def zc(i, carry):
        cnt_v[pl.ds(i * L, L)] = jnp.zeros((L,), jnp.float32)
        return carry
    lax.fori_loop(0, (NPAD + L) // L, zc, 0)
    plsc.subcore_barrier()

    ones = jnp.ones((L,), jnp.float32)

    def counts(k):
        for i in range(CH // L):
            plsc.addupdate_scatter(cnt_v, [dst_v[k, pl.ds(i * L, L)]], ones)

    def wait(sem):
        pltpu.make_async_copy(x_hbm.at[pl.ds(0, CH)], buf0, sem).wait()

    def gissue(k, buf, sem):
        pltpu.async_copy(x_hbm.at[src_v.at[k]], buf, sem)

    for blk in range(CPT // IB):
        pltpu.sync_copy(srcs_hbm.at[wid, pl.ds(blk * IB, IB)], src_v)
        pltpu.sync_copy(dsts_hbm.at[wid, pl.ds(blk * IB, IB)], dst_v)
        gissue(0, buf0, g0)

        def pair(i, carry):
            a = 2 * i
            wait(g0)
            gissue(a + 1, buf1, g1)
            counts(a)
            pltpu.sync_copy(buf0, acc_sh.at[dst_v.at[a]], add=True)
            gissue(a + 2, buf0, g0)
            wait(g1)
            counts(a + 1)
            pltpu.sync_copy(buf1, acc_sh.at[dst_v.at[a + 1]], add=True)
            return carry
        lax.fori_loop(0, (IB - 1) // 2, pair, 0)

        wait(g0)
        counts(IB - 1)
        pltpu.sync_copy(buf0, acc_sh.at[dst_v.at[IB - 1]], add=True)

    plsc.subcore_barrier()
    for j in range(RPT // CH):
        off = pl.multiple_of(s * RPT + j * CH, 8)
        pltpu.sync_copy(acc_sh.at[pl.ds(off, CH)], buf0)
        pltpu.sync_copy(buf0, p_hbm.at[c, pl.ds(off, CH)])
    pltpu.sync_copy(cnt_v.at[pl.ds(0, NPAD)], cnt_hbm.at[wid])


_AGG_CNT = pl.kernel(
    _agg_body,
    out_type=[
        jax.ShapeDtypeStruct((NC, NPAD, D), jnp.float32),
        jax.ShapeDtypeStruct((NW, NPAD), jnp.float32),
    ],
    mesh=plsc.VectorSubcoreMesh(core_axis_name="c", subcore_axis_name="s"),
    scratch_types=[
        pltpu.VMEM((IB, CH), jnp.int32),
        pltpu.VMEM((IB, CH), jnp.int32),
        pltpu.VMEM((CH, D), jnp.float32),
        pltpu.VMEM((CH, D), jnp.float32),
        pltpu.VMEM((NPAD + L,), jnp.float32),
        pltpu.VMEM_SHARED((NPAD + 8, D), jnp.float32),
        pltpu.SemaphoreType.DMA,
        pltpu.SemaphoreType.DMA,
    ],
    compiler_params=pltpu.CompilerParams(
        use_tc_tiling_on_sc=False, needs_layout_passes=False),
)

BN = 1024


def _layer_body(p_ref, c_ref, x_ref, wl_ref, wr_ref, b_ref, f_ref, o_ref):
    psum = p_ref[0] + p_ref[1]
    cnt_col = lax.dot_general(
        c_ref[...], jnp.ones((NW, 1), jnp.float32),
        (((0,), (0,)), ((), ())),
        preferred_element_type=jnp.float32,
        precision=lax.Precision.HIGHEST,
    )
    mean = psum / jnp.maximum(cnt_col, 1.0)
    acc = (
        jnp.dot(mean, wl_ref[...], preferred_element_type=jnp.float32,
                precision=lax.Precision.HIGHEST)
        + jnp.dot(x_ref[...], wr_ref[...], preferred_element_type=jnp.float32,
                  precision=lax.Precision.HIGHEST)
        + b_ref[...]
    )
    o_ref[...] = jnp.maximum(acc, acc * f_ref[...])


_LAYER = pl.pallas_call(
    _layer_body,
    grid=(NPAD // BN,),
    in_specs=[
        pl.BlockSpec((NC, BN, D), lambda i: (0, i, 0)),
        pl.BlockSpec((NW, BN), lambda i: (0, i)),
        pl.BlockSpec((BN, D), lambda i: (i, 0)),
        pl.BlockSpec((D, D), lambda i: (0, 0)),
        pl.BlockSpec((D, D), lambda i: (0, 0)),
        pl.BlockSpec((1, D), lambda i: (0, 0)),
        pl.BlockSpec((1, D), lambda i: (0, 0)),
    ],
    out_specs=pl.BlockSpec((BN, D), lambda i: (i, 0)),
    out_shape=jax.ShapeDtypeStruct((N, D), jnp.float32),
)


def kernel(x, edge_index, Wl1, Wr1, b1, Wl2, Wr2, b2):
    src3 = jnp.pad(edge_index[0].reshape(NW, EPT),
                   ((0, 0), (0, PADE))).reshape(NW, CPT, CH)
    dst3 = jnp.pad(edge_index[1].reshape(NW, EPT), ((0, 0), (0, PADE)),
                   constant_values=DUMP).reshape(NW, CPT, CH)
    Wl = jnp.stack([Wl1, Wl2])
    Wr = jnp.stack([Wr1, Wr2])
    bb = jnp.stack([b1.reshape(1, D), b2.reshape(1, D)])
    ff = jnp.stack([jnp.zeros((1, D), jnp.float32),
                    jnp.ones((1, D), jnp.float32)])

    def step(feat, ws):
        wl, wr, b, f = ws
        p, cnt = _AGG_CNT(feat, src3, dst3)
        return _LAYER(p, cnt, feat, wl, wr, b, f), 0.0

    out, _ = lax.scan(step, x, (Wl, Wr, bb, ff))
    return out


# speedup vs baseline: 2.8024x; 1.1255x over previous
import functools

import jax
import jax.numpy as jnp
from jax import lax
from jax.experimental import pallas as pl
from jax.experimental.pallas import tpu as pltpu
from jax.experimental.pallas import tpu_sc as plsc

N = 10000
D = 128
E = 320000
CH = 80
NC = 2
NS = 16
NW = NC * NS
EPT = E // NW
CPT = 125
PADE = CPT * CH - EPT
IB = 25
NPAD = 10240
DUMP = NPAD
RPT = NPAD // NS
L = 16


def _agg_body(x_hbm, srcs_hbm, dsts_hbm, p_hbm, cnt_hbm, src_v, dst_v,
              buf0, buf1, buf2, cnt_v, acc_sh, g0, g1, g2):
    c = lax.axis_index("c")
    s = lax.axis_index("s")
    wid = c * NS + s

    def zrow(i, carry):
        buf0[i // 8, pl.ds((i % 8) * L, L)] = jnp.zeros((L,), jnp.float32)
        return carry
    lax.fori_loop(0, CH * 8, zrow, 0)
    for j in range(RPT // CH):
        off = pl.multiple_of(s * RPT + j * CH, 8)
        pltpu.sync_copy(buf0, acc_sh.at[pl.ds(off, CH)])

    def zc(i, carry):
        cnt_v[pl.ds(i * L, L)] = jnp.zeros((L,), jnp.float32)
        return carry
    lax.fori_loop(0, (NPAD + L) // L, zc, 0)
    plsc.subcore_barrier()

    ones = jnp.ones((L,), jnp.float32)

    def counts(k):
        for i in range(CH // L):
            plsc.addupdate_scatter(cnt_v, [dst_v[k, pl.ds(i * L, L)]], ones)

    def wait(sem):
        pltpu.make_async_copy(x_hbm.at[pl.ds(0, CH)], buf0, sem).wait()

    def gissue(k, buf, sem):
        pltpu.async_copy(x_hbm.at[src_v.at[k]], buf, sem)

    bufs = (buf0, buf1, buf2)
    sems = (g0, g1, g2)

    def consume(j, t):
        wait(sems[t])
        counts(j)
        pltpu.sync_copy(bufs[t], acc_sh.at[dst_v.at[j]], add=True)

    for blk in range(CPT // IB):
        pltpu.sync_copy(srcs_hbm.at[wid, pl.ds(blk * IB, IB)], src_v)
        pltpu.sync_copy(dsts_hbm.at[wid, pl.ds(blk * IB, IB)], dst_v)
        gissue(0, buf0, g0)
        gissue(1, buf1, g1)
        gissue(2, buf2, g2)

        def triple(i, carry):
            for t in range(3):
                j = 3 * i + t
                consume(j, t)
                gissue(j + 3, bufs[t], sems[t])
            return carry
        lax.fori_loop(0, IB // 3 - 1, triple, 0)

        consume(IB - 4, 0)
        gissue(IB - 1, buf0, g0)
        consume(IB - 3, 1)
        consume(IB - 2, 2)
        consume(IB - 1, 0)

    plsc.subcore_barrier()
    for j in range(RPT // CH):
        off = pl.multiple_of(s * RPT + j * CH, 8)
        pltpu.sync_copy(acc_sh.at[pl.ds(off, CH)], buf0)
        pltpu.sync_copy(buf0, p_hbm.at[c, pl.ds(off, CH)])
    pltpu.sync_copy(cnt_v.at[pl.ds(0, NPAD)], cnt_hbm.at[wid])


_AGG_CNT = pl.kernel(
    _agg_body,
    out_type=[
        jax.ShapeDtypeStruct((NC, NPAD, D), jnp.float32),
        jax.ShapeDtypeStruct((NW, NPAD), jnp.float32),
    ],
    mesh=plsc.VectorSubcoreMesh(core_axis_name="c", subcore_axis_name="s"),
    scratch_types=[
        pltpu.VMEM((IB, CH), jnp.int32),
        pltpu.VMEM((IB, CH), jnp.int32),
        pltpu.VMEM((CH, D), jnp.float32),
        pltpu.VMEM((CH, D), jnp.float32),
        pltpu.VMEM((CH, D), jnp.float32),
        pltpu.VMEM((NPAD + L,), jnp.float32),
        pltpu.VMEM_SHARED((NPAD + 8, D), jnp.float32),
        pltpu.SemaphoreType.DMA,
        pltpu.SemaphoreType.DMA,
        pltpu.SemaphoreType.DMA,
    ],
    compiler_params=pltpu.CompilerParams(
        use_tc_tiling_on_sc=False, needs_layout_passes=False),
)

BN = 1024


def _layer_body(p_ref, c_ref, x_ref, wl_ref, wr_ref, b_ref, f_ref, o_ref):
    psum = p_ref[0] + p_ref[1]
    cnt_col = lax.dot_general(
        c_ref[...], jnp.ones((NW, 1), jnp.float32),
        (((0,), (0,)), ((), ())),
        preferred_element_type=jnp.float32,
        precision=lax.Precision.HIGHEST,
    )
    mean = psum / jnp.maximum(cnt_col, 1.0)
    acc = (
        jnp.dot(mean, wl_ref[...], preferred_element_type=jnp.float32,
                precision=lax.Precision.HIGHEST)
        + jnp.dot(x_ref[...], wr_ref[...], preferred_element_type=jnp.float32,
                  precision=lax.Precision.HIGHEST)
        + b_ref[...]
    )
    o_ref[...] = jnp.maximum(acc, acc * f_ref[...])


_LAYER = pl.pallas_call(
    _layer_body,
    grid=(NPAD // BN,),
    in_specs=[
        pl.BlockSpec((NC, BN, D), lambda i: (0, i, 0)),
        pl.BlockSpec((NW, BN), lambda i: (0, i)),
        pl.BlockSpec((BN, D), lambda i: (i, 0)),
        pl.BlockSpec((D, D), lambda i: (0, 0)),
        pl.BlockSpec((D, D), lambda i: (0, 0)),
        pl.BlockSpec((1, D), lambda i: (0, 0)),
        pl.BlockSpec((1, D), lambda i: (0, 0)),
    ],
    out_specs=pl.BlockSpec((BN, D), lambda i: (i, 0)),
    out_shape=jax.ShapeDtypeStruct((N, D), jnp.float32),
)


def kernel(x, edge_index, Wl1, Wr1, b1, Wl2, Wr2, b2):
    src3 = jnp.pad(edge_index[0].reshape(NW, EPT),
                   ((0, 0), (0, PADE))).reshape(NW, CPT, CH)
    dst3 = jnp.pad(edge_index[1].reshape(NW, EPT), ((0, 0), (0, PADE)),
                   constant_values=DUMP).reshape(NW, CPT, CH)
    Wl = jnp.stack([Wl1, Wl2])
    Wr = jnp.stack([Wr1, Wr2])
    bb = jnp.stack([b1.reshape(1, D), b2.reshape(1, D)])
    ff = jnp.stack([jnp.zeros((1, D), jnp.float32),
                    jnp.ones((1, D), jnp.float32)])

    def step(feat, ws):
        wl, wr, b, f = ws
        p, cnt = _AGG_CNT(feat, src3, dst3)
        return _LAYER(p, cnt, feat, wl, wr, b, f), 0.0

    out, _ = lax.scan(step, x, (Wl, Wr, bb, ff))
    return out


# speedup vs baseline: 2.9295x; 1.0454x over previous
import functools

import jax
import jax.numpy as jnp
from jax import lax
from jax.experimental import pallas as pl
from jax.experimental.pallas import tpu as pltpu
from jax.experimental.pallas import tpu_sc as plsc

N = 10000
D = 128
E = 320000
CH = 80
NC = 2
NS = 16
NW = NC * NS
EPT = E // NW
CPT = 125
PADE = CPT * CH - EPT
IB = 25
NPAD = 10240
DUMP = NPAD
RPT = NPAD // NS
L = 16


def _agg_body(x_hbm, ei_hbm, p_hbm, cnt_hbm, src_v, dst_v,
              buf0, buf1, buf2, cnt_v, acc_sh, g0, g1, g2):
    c = lax.axis_index("c")
    s = lax.axis_index("s")
    wid = c * NS + s

    def zrow(i, carry):
        buf0[i // 8, pl.ds((i % 8) * L, L)] = jnp.zeros((L,), jnp.float32)
        return carry
    lax.fori_loop(0, CH * 8, zrow, 0)
    for j in range(RPT // CH):
        off = pl.multiple_of(s * RPT + j * CH, 8)
        pltpu.sync_copy(buf0, acc_sh.at[pl.ds(off, CH)])

    def zc(i, carry):
        cnt_v[pl.ds(i * L, L)] = jnp.zeros((L,), jnp.float32)
        return carry
    lax.fori_loop(0, (NPAD + L) // L, zc, 0)
    plsc.subcore_barrier()

    ones = jnp.ones((L,), jnp.float32)

    def counts(k):
        for i in range(CH // L):
            plsc.addupdate_scatter(cnt_v, [dst_v[k, pl.ds(i * L, L)]], ones)

    def wait(sem):
        pltpu.make_async_copy(x_hbm.at[pl.ds(0, CH)], buf0, sem).wait()

    def gissue(k, buf, sem):
        pltpu.async_copy(x_hbm.at[src_v.at[k]], buf, sem)

    bufs = (buf0, buf1, buf2)
    sems = (g0, g1, g2)

    def consume(j, t):
        wait(sems[t])
        counts(j)
        pltpu.sync_copy(bufs[t], acc_sh.at[dst_v.at[j]], add=True)

    for blk in range(CPT // IB):
        pltpu.sync_copy(ei_hbm.at[0, wid, pl.ds(blk * IB, IB)], src_v)
        pltpu.sync_copy(ei_hbm.at[1, wid, pl.ds(blk * IB, IB)], dst_v)
        gissue(0, buf0, g0)
        gissue(1, buf1, g1)
        gissue(2, buf2, g2)

        def triple(i, carry):
            for t in range(3):
                j = 3 * i + t
                consume(j, t)
                gissue(j + 3, bufs[t], sems[t])
            return carry
        lax.fori_loop(0, IB // 3 - 1, triple, 0)

        consume(IB - 4, 0)
        gissue(IB - 1, buf0, g0)
        consume(IB - 3, 1)
        consume(IB - 2, 2)
        consume(IB - 1, 0)

    plsc.subcore_barrier()
    for j in range(RPT // CH):
        off = pl.multiple_of(s * RPT + j * CH, 8)
        pltpu.sync_copy(acc_sh.at[pl.ds(off, CH)], buf0)
        pltpu.sync_copy(buf0, p_hbm.at[c, pl.ds(off, CH)])
    pltpu.sync_copy(cnt_v.at[pl.ds(0, NPAD)], cnt_hbm.at[wid])


_AGG_CNT = pl.kernel(
    _agg_body,
    out_type=[
        jax.ShapeDtypeStruct((NC, NPAD, D), jnp.float32),
        jax.ShapeDtypeStruct((NW, NPAD), jnp.float32),
    ],
    mesh=plsc.VectorSubcoreMesh(core_axis_name="c", subcore_axis_name="s"),
    scratch_types=[
        pltpu.VMEM((IB, CH), jnp.int32),
        pltpu.VMEM((IB, CH), jnp.int32),
        pltpu.VMEM((CH, D), jnp.float32),
        pltpu.VMEM((CH, D), jnp.float32),
        pltpu.VMEM((CH, D), jnp.float32),
        pltpu.VMEM((NPAD + L,), jnp.float32),
        pltpu.VMEM_SHARED((NPAD + 8, D), jnp.float32),
        pltpu.SemaphoreType.DMA,
        pltpu.SemaphoreType.DMA,
        pltpu.SemaphoreType.DMA,
    ],
    compiler_params=pltpu.CompilerParams(
        use_tc_tiling_on_sc=False, needs_layout_passes=False),
)

BN = 2048


def _layer_body(p_ref, c_ref, x_ref, wl_ref, wr_ref, b_ref, f_ref, o_ref):
    psum = p_ref[0] + p_ref[1]
    cnt_col = lax.dot_general(
        c_ref[...], jnp.ones((NW, 1), jnp.float32),
        (((0,), (0,)), ((), ())),
        preferred_element_type=jnp.float32,
        precision=lax.Precision.HIGHEST,
    )
    mean = psum / jnp.maximum(cnt_col, 1.0)
    acc = (
        jnp.dot(mean, wl_ref[...], preferred_element_type=jnp.float32,
                precision=lax.Precision.HIGHEST)
        + jnp.dot(x_ref[...], wr_ref[...], preferred_element_type=jnp.float32,
                  precision=lax.Precision.HIGHEST)
        + b_ref[...]
    )
    o_ref[...] = jnp.maximum(acc, acc * f_ref[...])


_LAYER = pl.pallas_call(
    _layer_body,
    grid=(NPAD // BN,),
    in_specs=[
        pl.BlockSpec((NC, BN, D), lambda i: (0, i, 0)),
        pl.BlockSpec((NW, BN), lambda i: (0, i)),
        pl.BlockSpec((BN, D), lambda i: (i, 0)),
        pl.BlockSpec((D, D), lambda i: (0, 0)),
        pl.BlockSpec((D, D), lambda i: (0, 0)),
        pl.BlockSpec((1, D), lambda i: (0, 0)),
        pl.BlockSpec((1, D), lambda i: (0, 0)),
    ],
    out_specs=pl.BlockSpec((BN, D), lambda i: (i, 0)),
    out_shape=jax.ShapeDtypeStruct((N, D), jnp.float32),
)


def kernel(x, edge_index, Wl1, Wr1, b1, Wl2, Wr2, b2):
    ei4 = edge_index.reshape(2, NW, CPT, CH)
    Wl = jnp.stack([Wl1, Wl2])
    Wr = jnp.stack([Wr1, Wr2])
    bb = jnp.stack([b1.reshape(1, D), b2.reshape(1, D)])
    ff = jnp.stack([jnp.zeros((1, D), jnp.float32),
                    jnp.ones((1, D), jnp.float32)])

    def step(feat, ws):
        wl, wr, b, f = ws
        p, cnt = _AGG_CNT(feat, ei4)
        return _LAYER(p, cnt, feat, wl, wr, b, f), 0.0

    out, _ = lax.scan(step, x, (Wl, Wr, bb, ff))
    return out


# speedup vs baseline: 3.0966x; 1.0570x over previous
import functools

import jax
import jax.numpy as jnp
from jax import lax
from jax.experimental import pallas as pl
from jax.experimental.pallas import tpu as pltpu
from jax.experimental.pallas import tpu_sc as plsc

N = 10000
D = 128
E = 320000
CH = 80
NC = 2
NS = 16
NW = NC * NS
EPT = E // NW
CPT = 125
PADE = CPT * CH - EPT
IB = 25
NPAD = 10240
DUMP = NPAD
RPT = NPAD // NS
L = 16


def _agg_body(x_hbm, ei_hbm, p_hbm, cnt_hbm, src_v, dst_v,
              buf0, buf1, buf2, cnt_v, acc_sh, g0, g1, g2):
    c = lax.axis_index("c")
    s = lax.axis_index("s")
    wid = c * NS + s

    def zrow(i, carry):
        buf0[i // 8, pl.ds((i % 8) * L, L)] = jnp.zeros((L,), jnp.float32)
        return carry
    lax.fori_loop(0, CH * 8, zrow, 0)
    for j in range(RPT // CH):
        off = pl.multiple_of(s * RPT + j * CH, 8)
        pltpu.sync_copy(buf0, acc_sh.at[pl.ds(off, CH)])

    def zc(i, carry):
        cnt_v[pl.ds(i * L, L)] = jnp.zeros((L,), jnp.float32)
        return carry
    lax.fori_loop(0, (NPAD + L) // L, zc, 0)
    plsc.subcore_barrier()

    ones = jnp.ones((L,), jnp.float32)

    def counts(k):
        for i in range(CH // L):
            plsc.addupdate_scatter(cnt_v, [dst_v[k, pl.ds(i * L, L)]], ones)

    def wait(sem):
        pltpu.make_async_copy(x_hbm.at[pl.ds(0, CH)], buf0, sem).wait()

    def gissue(k, buf, sem):
        pltpu.async_copy(x_hbm.at[src_v.at[k]], buf, sem)

    bufs = (buf0, buf1, buf2)
    sems = (g0, g1, g2)

    def consume(j, t):
        wait(sems[t])
        counts(j)
        pltpu.sync_copy(bufs[t], acc_sh.at[dst_v.at[j]], add=True)

    for blk in range(CPT // IB):
        pltpu.sync_copy(ei_hbm.at[0, wid, pl.ds(blk * IB, IB)], src_v)
        pltpu.sync_copy(ei_hbm.at[1, wid, pl.ds(blk * IB, IB)], dst_v)
        gissue(0, buf0, g0)
        gissue(1, buf1, g1)
        gissue(2, buf2, g2)

        def triple(i, carry):
            for t in range(3):
                j = 3 * i + t
                consume(j, t)
                gissue(j + 3, bufs[t], sems[t])
            return carry
        lax.fori_loop(0, IB // 3 - 1, triple, 0)

        consume(IB - 4, 0)
        gissue(IB - 1, buf0, g0)
        consume(IB - 3, 1)
        consume(IB - 2, 2)
        consume(IB - 1, 0)

    plsc.subcore_barrier()
    for j in range(RPT // CH):
        off = pl.multiple_of(s * RPT + j * CH, 8)
        pltpu.sync_copy(acc_sh.at[pl.ds(off, CH)], buf0)
        pltpu.sync_copy(buf0, p_hbm.at[c, pl.ds(off, CH)])
    pltpu.sync_copy(cnt_v.at[pl.ds(0, NPAD)], cnt_hbm.at[wid])


_AGG_CNT = pl.kernel(
    _agg_body,
    out_type=[
        jax.ShapeDtypeStruct((NC, NPAD, D), jnp.float32),
        jax.ShapeDtypeStruct((NW, NPAD), jnp.float32),
    ],
    mesh=plsc.VectorSubcoreMesh(core_axis_name="c", subcore_axis_name="s"),
    scratch_types=[
        pltpu.VMEM((IB, CH), jnp.int32),
        pltpu.VMEM((IB, CH), jnp.int32),
        pltpu.VMEM((CH, D), jnp.float32),
        pltpu.VMEM((CH, D), jnp.float32),
        pltpu.VMEM((CH, D), jnp.float32),
        pltpu.VMEM((NPAD + L,), jnp.float32),
        pltpu.VMEM_SHARED((NPAD + 8, D), jnp.float32),
        pltpu.SemaphoreType.DMA,
        pltpu.SemaphoreType.DMA,
        pltpu.SemaphoreType.DMA,
    ],
    compiler_params=pltpu.CompilerParams(
        use_tc_tiling_on_sc=False, needs_layout_passes=False),
)

BN = 2048


def _layer_body(p_ref, c_ref, x_ref, wl_ref, wr_ref, b_ref, f_ref, o_ref):
    psum = p_ref[0] + p_ref[1]
    cnt_col = lax.dot_general(
        c_ref[...], jnp.ones((NW, 1), jnp.float32),
        (((0,), (0,)), ((), ())),
        preferred_element_type=jnp.float32,
        precision=lax.Precision.HIGHEST,
    )
    mean = psum / jnp.maximum(cnt_col, 1.0)
    acc = (
        jnp.dot(mean, wl_ref[...], preferred_element_type=jnp.float32)
        + jnp.dot(x_ref[...], wr_ref[...], preferred_element_type=jnp.float32)
        + b_ref[...]
    )
    o_ref[...] = jnp.maximum(acc, acc * f_ref[...])


_LAYER = pl.pallas_call(
    _layer_body,
    grid=(NPAD // BN,),
    in_specs=[
        pl.BlockSpec((NC, BN, D), lambda i: (0, i, 0)),
        pl.BlockSpec((NW, BN), lambda i: (0, i)),
        pl.BlockSpec((BN, D), lambda i: (i, 0)),
        pl.BlockSpec((D, D), lambda i: (0, 0)),
        pl.BlockSpec((D, D), lambda i: (0, 0)),
        pl.BlockSpec((1, D), lambda i: (0, 0)),
        pl.BlockSpec((1, D), lambda i: (0, 0)),
    ],
    out_specs=pl.BlockSpec((BN, D), lambda i: (i, 0)),
    out_shape=jax.ShapeDtypeStruct((N, D), jnp.float32),
)


def kernel(x, edge_index, Wl1, Wr1, b1, Wl2, Wr2, b2):
    ei4 = edge_index.reshape(2, NW, CPT, CH)
    Wl = jnp.stack([Wl1, Wl2])
    Wr = jnp.stack([Wr1, Wr2])
    bb = jnp.stack([b1.reshape(1, D), b2.reshape(1, D)])
    ff = jnp.stack([jnp.zeros((1, D), jnp.float32),
                    jnp.ones((1, D), jnp.float32)])

    def step(feat, ws):
        wl, wr, b, f = ws
        p, cnt = _AGG_CNT(feat, ei4)
        return _LAYER(p, cnt, feat, wl, wr, b, f), 0.0

    out, _ = lax.scan(step, x, (Wl, Wr, bb, ff))
    return out


# speedup vs baseline: 3.3447x; 1.0801x over previous
import functools

import jax
import jax.numpy as jnp
from jax import lax
from jax.experimental import pallas as pl
from jax.experimental.pallas import tpu as pltpu
from jax.experimental.pallas import tpu_sc as plsc

N = 10000
D = 128
E = 320000
CH = 80
NC = 2
NS = 16
NW = NC * NS
EPT = E // NW
CPT = 125
PADE = CPT * CH - EPT
IB = 25
NPAD = 10240
DUMP = NPAD
RPT = NPAD // NS
L = 16


def _agg_body(x_hbm, ei_hbm, p_hbm, cnt_hbm, srcA, dstA, srcB, dstB,
              buf0, buf1, buf2, cnt_v, acc_sh, g0, g1, g2, gI):
    c = lax.axis_index("c")
    s = lax.axis_index("s")
    wid = c * NS + s

    def zrow(i, carry):
        buf0[i // 8, pl.ds((i % 8) * L, L)] = jnp.zeros((L,), jnp.float32)
        return carry
    lax.fori_loop(0, CH * 8, zrow, 0)
    for j in range(RPT // CH):
        off = pl.multiple_of(s * RPT + j * CH, 8)
        pltpu.async_copy(buf0, acc_sh.at[pl.ds(off, CH)], g0)

    def zc(i, carry):
        cnt_v[pl.ds(i * L, L)] = jnp.zeros((L,), jnp.float32)
        return carry
    lax.fori_loop(0, (NPAD + L) // L, zc, 0)
    pltpu.async_copy(ei_hbm.at[0, wid, pl.ds(0, IB)], srcA, gI)
    pltpu.async_copy(ei_hbm.at[1, wid, pl.ds(0, IB)], dstA, gI)
    for j in range(RPT // CH):
        pltpu.make_async_copy(buf0, acc_sh.at[pl.ds(0, CH)], g0).wait()
    pltpu.make_async_copy(ei_hbm.at[0, wid, pl.ds(0, IB)], srcA, gI).wait()
    pltpu.make_async_copy(ei_hbm.at[1, wid, pl.ds(0, IB)], dstA, gI).wait()
    plsc.subcore_barrier()

    ones = jnp.ones((L,), jnp.float32)
    bufs = (buf0, buf1, buf2)
    sems = (g0, g1, g2)

    def wait(sem):
        pltpu.make_async_copy(x_hbm.at[pl.ds(0, CH)], buf0, sem).wait()

    for blk in range(CPT // IB):
        src_v, dst_v = (srcA, dstA) if blk % 2 == 0 else (srcB, dstB)
        if blk + 1 < CPT // IB:
            nsrc, ndst = (srcB, dstB) if blk % 2 == 0 else (srcA, dstA)
            pltpu.async_copy(ei_hbm.at[0, wid, pl.ds((blk + 1) * IB, IB)],
                             nsrc, gI)
            pltpu.async_copy(ei_hbm.at[1, wid, pl.ds((blk + 1) * IB, IB)],
                             ndst, gI)

        def counts(k):
            for i in range(CH // L):
                plsc.addupdate_scatter(cnt_v, [dst_v[k, pl.ds(i * L, L)]],
                                       ones)

        def gissue(k, buf, sem):
            pltpu.async_copy(x_hbm.at[src_v.at[k]], buf, sem)

        def consume(j, t):
            wait(sems[t])
            counts(j)
            pltpu.sync_copy(bufs[t], acc_sh.at[dst_v.at[j]], add=True)

        gissue(0, buf0, g0)
        gissue(1, buf1, g1)
        gissue(2, buf2, g2)

        def triple(i, carry):
            for t in range(3):
                j = 3 * i + t
                consume(j, t)
                gissue(j + 3, bufs[t], sems[t])
            return carry
        lax.fori_loop(0, IB // 3 - 1, triple, 0)

        consume(IB - 4, 0)
        gissue(IB - 1, buf0, g0)
        consume(IB - 3, 1)
        consume(IB - 2, 2)
        consume(IB - 1, 0)
        if blk + 1 < CPT // IB:
            pltpu.make_async_copy(ei_hbm.at[0, wid, pl.ds(0, IB)], nsrc,
                                  gI).wait()
            pltpu.make_async_copy(ei_hbm.at[1, wid, pl.ds(0, IB)], ndst,
                                  gI).wait()

    plsc.subcore_barrier()
    def roff(j):
        return pl.multiple_of(s * RPT + j * CH, 8)
    pltpu.async_copy(acc_sh.at[pl.ds(roff(0), CH)], buf0, g0)
    for j in range(RPT // CH):
        b, sem = (buf0, g0) if j % 2 == 0 else (buf1, g1)
        pltpu.make_async_copy(acc_sh.at[pl.ds(roff(j), CH)], b, sem).wait()
        if j + 1 < RPT // CH:
            nb, nsem = (buf1, g1) if j % 2 == 0 else (buf0, g0)
            pltpu.async_copy(acc_sh.at[pl.ds(roff(j + 1), CH)], nb, nsem)
        pltpu.sync_copy(b, p_hbm.at[c, pl.ds(roff(j), CH)])
    pltpu.sync_copy(cnt_v.at[pl.ds(0, NPAD)], cnt_hbm.at[wid])


_AGG_CNT = pl.kernel(
    _agg_body,
    out_type=[
        jax.ShapeDtypeStruct((NC, NPAD, D), jnp.float32),
        jax.ShapeDtypeStruct((NW, NPAD), jnp.float32),
    ],
    mesh=plsc.VectorSubcoreMesh(core_axis_name="c", subcore_axis_name="s"),
    scratch_types=[
        pltpu.VMEM((IB, CH), jnp.int32),
        pltpu.VMEM((IB, CH), jnp.int32),
        pltpu.VMEM((IB, CH), jnp.int32),
        pltpu.VMEM((IB, CH), jnp.int32),
        pltpu.VMEM((CH, D), jnp.float32),
        pltpu.VMEM((CH, D), jnp.float32),
        pltpu.VMEM((CH, D), jnp.float32),
        pltpu.VMEM((NPAD + L,), jnp.float32),
        pltpu.VMEM_SHARED((NPAD + 8, D), jnp.float32),
        pltpu.SemaphoreType.DMA,
        pltpu.SemaphoreType.DMA,
        pltpu.SemaphoreType.DMA,
        pltpu.SemaphoreType.DMA,
    ],
    compiler_params=pltpu.CompilerParams(
        use_tc_tiling_on_sc=False, needs_layout_passes=False),
)

BN = 2048


def _layer_body(p_ref, c_ref, x_ref, wl_ref, wr_ref, b_ref, f_ref, o_ref):
    psum = p_ref[0] + p_ref[1]
    cnt_col = lax.dot_general(
        c_ref[...], jnp.ones((NW, 1), jnp.float32),
        (((0,), (0,)), ((), ())),
        preferred_element_type=jnp.float32,
        precision=lax.Precision.HIGHEST,
    )
    mean = psum / jnp.maximum(cnt_col, 1.0)
    acc = (
        jnp.dot(mean, wl_ref[...], preferred_element_type=jnp.float32)
        + jnp.dot(x_ref[...], wr_ref[...], preferred_element_type=jnp.float32)
        + b_ref[...]
    )
    o_ref[...] = jnp.maximum(acc, acc * f_ref[...])


_LAYER = pl.pallas_call(
    _layer_body,
    grid=(NPAD // BN,),
    in_specs=[
        pl.BlockSpec((NC, BN, D), lambda i: (0, i, 0)),
        pl.BlockSpec((NW, BN), lambda i: (0, i)),
        pl.BlockSpec((BN, D), lambda i: (i, 0)),
        pl.BlockSpec((D, D), lambda i: (0, 0)),
        pl.BlockSpec((D, D), lambda i: (0, 0)),
        pl.BlockSpec((1, D), lambda i: (0, 0)),
        pl.BlockSpec((1, D), lambda i: (0, 0)),
    ],
    out_specs=pl.BlockSpec((BN, D), lambda i: (i, 0)),
    out_shape=jax.ShapeDtypeStruct((N, D), jnp.float32),
)


def kernel(x, edge_index, Wl1, Wr1, b1, Wl2, Wr2, b2):
    ei4 = edge_index.reshape(2, NW, CPT, CH)
    Wl = jnp.stack([Wl1, Wl2])
    Wr = jnp.stack([Wr1, Wr2])
    bb = jnp.stack([b1.reshape(1, D), b2.reshape(1, D)])
    ff = jnp.stack([jnp.zeros((1, D), jnp.float32),
                    jnp.ones((1, D), jnp.float32)])

    def step(feat, ws):
        wl, wr, b, f = ws
        p, cnt = _AGG_CNT(feat, ei4)
        return _LAYER(p, cnt, feat, wl, wr, b, f), 0.0

    out, _ = lax.scan(step, x, (Wl, Wr, bb, ff))
    return out
